# baseline (speedup 1.0000x reference)
import functools

import jax
import jax.numpy as jnp
import numpy as np
from jax import lax
from jax.experimental import pallas as pl
from jax.experimental.pallas import tpu as pltpu
from jax.experimental.pallas import tpu_sc as plsc

L = 512
E = 16384
D_NODE = 256
D_PAIR = 128
L0 = 32
L1 = 8
BE = 2048
G = E // BE
NW = 32
EPW = E // NW
GC = 4
CW = EPW // GC

_N0 = 1.0 / np.sqrt(40.0)
_N1 = np.sqrt(3.0 / 48.0)
_I3 = 1.0 / np.sqrt(3.0)


def _np_c121():
    C = np.zeros((3, 3, 5), dtype=np.float32)
    c = 1.0 / np.sqrt(10.0)
    d = 1.0 / np.sqrt(30.0)
    C[0, 2, 0] = c; C[2, 0, 0] = c
    C[0, 1, 1] = c; C[1, 0, 1] = c
    C[1, 1, 2] = 2.0 * d; C[0, 0, 2] = -d; C[2, 2, 2] = -d
    C[1, 2, 3] = c; C[2, 1, 3] = c
    C[2, 2, 4] = c; C[0, 0, 4] = -c
    return np.transpose(C, (2, 0, 1)).reshape(5, 9)


def _f32(x):
    return jnp.asarray(np.asarray(x, dtype=np.float32))


def _bf16(x):
    return jnp.asarray(np.asarray(x, dtype=np.float32).astype(jnp.bfloat16))


_E = np.eye
_O = np.ones
_R_EXP_32_1024 = _bf16(np.kron(_E(32), _O((1, 32))))
_R_FOLD_1024_32 = _f32(_N0 * np.tile(_E(32), (32, 1)))
_R_EXP_32_256 = _f32(np.kron(_E(32), _O((1, 8))))
_R_FOLD_256_8 = _f32(np.tile(_E(8), (32, 1)))
_R_EXP_8_256 = _f32(np.kron(_E(8), _O((1, 32))))
_R_FOLD_256_32 = _f32(_N0 * _I3 * np.tile(_E(32), (8, 1)))
_R_M_24 = _f32(np.tile(_E(3), (1, 8)))
_R_FOLD_24_8 = _f32(np.kron(_E(8), _O((3, 1))))
_R_V_24 = _f32(_N1 * _I3 * np.kron(_E(8), _O((1, 3))))
_R_A_64_192 = _f32(np.kron(_E(64), _O((1, 3))))
_R_B_24_192 = _f32(np.kron(_E(8), np.tile(_E(3), (1, 8))))
_R_F_192_24A = _f32(_N1 * _I3 * np.tile(_E(24), (8, 1)))
_R_F_192_24B = _f32(_N1 * np.tile(_E(24), (8, 1)))
_R_A_24_72 = _f32(np.kron(_E(24), _O((1, 3))))
_R_B_9_72 = _f32(np.tile(_E(9), (1, 8)))
_R_F_72_24 = _f32(np.kron(_E(8), np.tile(_E(3), (3, 1))))
_CD = _f32(_np_c121())
_EYE3 = _f32(np.eye(3))


@functools.cache
def _build_sc_gather():
    @functools.partial(
        pl.kernel,
        mesh=plsc.VectorSubcoreMesh(core_axis_name="c", subcore_axis_name="s"),
        out_type=jax.ShapeDtypeStruct((E, D_PAIR), jnp.float32),
        scratch_types=[
            pltpu.VMEM((EPW,), jnp.int32),
            pltpu.VMEM((EPW,), jnp.int32),
            pltpu.VMEM((GC, CW), jnp.int32),
            pltpu.VMEM((GC, CW, D_PAIR), jnp.float32),
            pltpu.SemaphoreType.DMA,
        ],
    )
    def _sc_gather(ii_hbm, jj_hbm, table_hbm, out_hbm, ii_v, jj_v, idx_v,
                   rows_v, sem):
        wid = lax.axis_index("s") * 2 + lax.axis_index("c")
        base = wid * EPW
        pltpu.sync_copy(ii_hbm.at[pl.ds(base, EPW)], ii_v)
        pltpu.sync_copy(jj_hbm.at[pl.ds(base, EPW)], jj_v)
        for c in range(GC):
            for k in range(CW // 16):
                s = pl.ds(c * CW + k * 16, 16)
                idx_v[c, pl.ds(k * 16, 16)] = ii_v[s] * L + jj_v[s]
        copies = [
            pltpu.async_copy(table_hbm.at[idx_v.at[c]], rows_v.at[c], sem)
            for c in range(GC)
        ]
        for cp in copies:
            cp.wait()
        for c in range(GC):
            pltpu.sync_copy(rows_v.at[c],
                            out_hbm.at[pl.ds(base + c * CW, CW)])

    return _sc_gather


def _tc_body(ef_ref, src_ref, dst_ref, sh_ref, node_ref, l1f_ref,
             plw_ref, plb_ref, lng_ref, lnb_ref, f1w_ref, f1b_ref,
             wmw_ref, pnw_ref, pnb_ref,
             cd_ref, e32w_ref, f1024_ref, e328_ref, f2568_ref, e832_ref,
             f25632_ref, rm24_ref, rf248_ref, rv24_ref, ra64_ref, rb24_ref,
             rf192a_ref, rf192b_ref, ra24_ref, rb9_ref, rf72_ref,
             b1_ref, b2_ref, b3_ref, b4_ref, b5_ref,
             nodeout_ref, l1o_ref,
             ng_s, acc0_s, acc1_s, cnt_s):
    g = pl.program_id(0)

    @pl.when(g == 0)
    def _init():
        ng_s[:, 0:L0] = jnp.dot(node_ref[:], plw_ref[:],
                                preferred_element_type=jnp.float32) + plb_ref[:]
        ng_s[:, L0:L0 + 3 * L1] = l1f_ref[:]
        acc0_s[:] = jnp.zeros_like(acc0_s)
        acc1_s[:] = jnp.zeros_like(acc1_s)
        cnt_s[:] = jnp.zeros_like(cnt_s)

    ef = ef_ref[:]
    mu = jnp.mean(ef, axis=1, keepdims=True)
    xc = ef - mu
    var = jnp.mean(xc * xc, axis=1, keepdims=True)
    h = xc * lax.rsqrt(var + 1e-5) * lng_ref[:] + lnb_ref[:]
    h = jnp.maximum(jnp.dot(h, f1w_ref[:],
                            preferred_element_type=jnp.float32) + f1b_ref[:],
                    0.0)

    wm = jnp.dot(h.astype(jnp.bfloat16), wmw_ref[:],
                 preferred_element_type=jnp.float32)
    w1m = wm[:, 0:1024]
    w2m = wm[:, 1024:1280]
    w3m = wm[:, 1280:1344]
    w4m = wm[:, 1408:1664]
    w5m = wm[:, 1664:1728]

    dstv = dst_ref[0]
    iota_n = lax.broadcasted_iota(jnp.int32, (BE, L), 1)
    ohd = (dstv == iota_n).astype(jnp.float32)
    xg = jnp.dot(ohd, ng_s[:], preferred_element_type=jnp.float32)
    xs = xg[:, 0:L0]
    xv = xg[:, L0:L0 + 3 * L1]

    sh = sh_ref[:]
    x2s = sh[:, 0:1]
    x2v = sh[:, 1:4]
    x2t = sh[:, 4:9]

    def md(a, b):
        return jnp.dot(a, b, preferred_element_type=jnp.float32)

    xs2 = xs * x2s
    rep1 = jnp.dot(xs2.astype(jnp.bfloat16), e32w_ref[:],
                   preferred_element_type=jnp.float32)
    o0 = md(w1m * rep1, f1024_ref[:]) + md(xs2, b1_ref[:])
    x2v24 = md(x2v, rm24_ref[:])
    dotp = md(xv * x2v24, rf248_ref[:])
    o0 = o0 + md(w4m * md(dotp, e832_ref[:]), f25632_ref[:]) \
            + md(dotp, b4_ref[:])
    t2 = md(w2m * md(xs, e328_ref[:]), f2568_ref[:]) + md(xs, b2_ref[:])
    out1 = md(t2, rv24_ref[:]) * x2v24
    xvs = xv * x2s
    out1 = out1 + md(md(w3m, ra64_ref[:]) * md(xvs, rb24_ref[:]),
                     rf192a_ref[:]) + md(xvs, b3_ref[:])
    dmat = md(x2t, cd_ref[:])
    tmp = md(md(xv, ra24_ref[:]) * md(dmat, rb9_ref[:]), rf72_ref[:])
    out1 = out1 + md(md(w5m, ra64_ref[:]) * md(tmp, rb24_ref[:]),
                     rf192b_ref[:]) + md(tmp, b5_ref[:])

    srow = src_ref[0]
    iota_t = lax.broadcasted_iota(jnp.int32, (L, BE), 0)
    ohsT = (srow == iota_t).astype(jnp.float32)
    acc0_s[:] = acc0_s[:] + md(ohsT, o0)
    acc1_s[:] = acc1_s[:] + md(ohsT, out1)
    cnt_s[:] = cnt_s[:] + jnp.sum(ohsT, axis=1, keepdims=True)

    @pl.when(g == G - 1)
    def _finish():
        cnt = jnp.maximum(cnt_s[:], 1.0)
        m0 = acc0_s[:] / cnt
        m1 = acc1_s[:] / cnt
        nodeout_ref[:] = (jnp.dot(m0, pnw_ref[:],
                                  preferred_element_type=jnp.float32)
                          + pnb_ref[:] + node_ref[:])
        l1o_ref[:] = m1 + l1f_ref[:]


def _rep(shape):
    nd = len(shape)
    return pl.BlockSpec(shape, lambda g, _n=nd: (0,) * _n)


def _build_tc(interpret: bool = False):
    in_specs = [
        pl.BlockSpec((BE, D_PAIR), lambda g: (g, 0)),
        pl.BlockSpec((1, 1, BE), lambda g: (g, 0, 0)),
        pl.BlockSpec((1, BE, 1), lambda g: (g, 0, 0)),
        pl.BlockSpec((BE, 9), lambda g: (g, 0)),
        _rep((L, D_NODE)),
        _rep((L, 3 * L1)),
        _rep((D_NODE, L0)), _rep((1, L0)),
        _rep((1, D_PAIR)), _rep((1, D_PAIR)),
        _rep((D_PAIR, D_PAIR)), _rep((1, D_PAIR)),
        _rep((D_PAIR, 1792)),
        _rep((L0, D_NODE)), _rep((1, D_NODE)),
        _rep((5, 9)),
        _rep((32, 1024)), _rep((1024, 32)),
        _rep((32, 256)), _rep((256, 8)),
        _rep((8, 256)), _rep((256, 32)),
        _rep((3, 24)), _rep((24, 8)), _rep((8, 24)),
        _rep((64, 192)), _rep((24, 192)),
        _rep((192, 24)), _rep((192, 24)),
        _rep((24, 72)), _rep((9, 72)), _rep((72, 24)),
        _rep((32, 32)), _rep((32, 8)), _rep((24, 24)),
        _rep((8, 32)), _rep((24, 24)),
    ]
    out_specs = [
        pl.BlockSpec((L, D_NODE), lambda g: (0, 0)),
        pl.BlockSpec((L, 3 * L1), lambda g: (0, 0)),
    ]
    return pl.pallas_call(
        _tc_body,
        grid=(G,),
        in_specs=in_specs,
        out_specs=out_specs,
        out_shape=[
            jax.ShapeDtypeStruct((L, D_NODE), jnp.float32),
            jax.ShapeDtypeStruct((L, 3 * L1), jnp.float32),
        ],
        scratch_shapes=[
            pltpu.VMEM((L, L0 + 3 * L1), jnp.float32),
            pltpu.VMEM((L, L0), jnp.float32),
            pltpu.VMEM((L, 3 * L1), jnp.float32),
            pltpu.VMEM((L, 1), jnp.float32),
        ],
        compiler_params=pltpu.CompilerParams(
            dimension_semantics=("arbitrary",)),
        interpret=interpret,
    )


def _pad_fc2(w):
    z = jnp.zeros((w.shape[0], 64), jnp.float32)
    return jnp.concatenate(
        [w[:, 0:1280], w[:, 1280:1344], z, w[:, 1344:1600],
         w[:, 1600:1664], z], axis=1)


def _prep_args(ef, node, l1_feats, edge_src, edge_dst, edge_sh,
               proj_l0_w, proj_l0_b, ln_g, ln_b, fc1_w, fc1_b, fc2_w, fc2_b,
               proj_node_w, proj_node_b):
    r1 = lambda a: a.reshape(1, -1)
    b1 = _N0 * fc2_b[0:1024].reshape(L0, L0)
    b2 = fc2_b[1024:1280].reshape(L0, L1)
    b3 = _N1 * _I3 * jnp.kron(fc2_b[1280:1344].reshape(L1, L1), _EYE3)
    b4 = _N0 * _I3 * fc2_b[1344:1600].reshape(L1, L0)
    b5 = _N1 * jnp.kron(fc2_b[1600:1664].reshape(L1, L1), _EYE3)
    return (
        ef,
        edge_src.reshape(G, 1, BE),
        edge_dst.reshape(G, BE, 1),
        edge_sh,
        node.reshape(L, D_NODE),
        l1_feats.reshape(L, 3 * L1),
        proj_l0_w, r1(proj_l0_b), r1(ln_g), r1(ln_b),
        fc1_w, r1(fc1_b),
        _pad_fc2(fc2_w).astype(jnp.bfloat16),
        proj_node_w, r1(proj_node_b),
        _CD,
        _R_EXP_32_1024, _R_FOLD_1024_32,
        _R_EXP_32_256, _R_FOLD_256_8,
        _R_EXP_8_256, _R_FOLD_256_32,
        _R_M_24, _R_FOLD_24_8, _R_V_24,
        _R_A_64_192, _R_B_24_192,
        _R_F_192_24A, _R_F_192_24B,
        _R_A_24_72, _R_B_9_72, _R_F_72_24,
        b1, b2, b3, b4, b5,
    )


def kernel(node, pair, l1_feats, pair_index, edge_src, edge_dst, edge_sh,
           proj_l0_w, proj_l0_b, ln_g, ln_b, fc1_w, fc1_b, fc2_w, fc2_b,
           proj_node_w, proj_node_b):
    table = pair.reshape(L * L, D_PAIR)
    ef = _build_sc_gather()(pair_index[1], pair_index[2], table)
    args = _prep_args(ef, node, l1_feats, edge_src, edge_dst, edge_sh,
                      proj_l0_w, proj_l0_b, ln_g, ln_b, fc1_w, fc1_b,
                      fc2_w, fc2_b, proj_node_w, proj_node_b)
    node_out, l1o = _build_tc()(*args)
    return (node_out.reshape(1, L, D_NODE), l1o.reshape(1, L, 3 * L1))


# speedup vs baseline: 1.0760x; 1.0760x over previous
import functools

import jax
import jax.numpy as jnp
import numpy as np
from jax import lax
from jax.experimental import pallas as pl
from jax.experimental.pallas import tpu as pltpu
from jax.experimental.pallas import tpu_sc as plsc

L = 512
E = 16384
D_NODE = 256
D_PAIR = 128
L0 = 32
L1 = 8
BE = 2048
G = E // BE
NW = 32
EPW = E // NW
GC = 4
CW = EPW // GC

_N0 = 1.0 / np.sqrt(40.0)
_N1 = np.sqrt(3.0 / 48.0)
_I3 = 1.0 / np.sqrt(3.0)


def _np_c121():
    C = np.zeros((3, 3, 5), dtype=np.float32)
    c = 1.0 / np.sqrt(10.0)
    d = 1.0 / np.sqrt(30.0)
    C[0, 2, 0] = c; C[2, 0, 0] = c
    C[0, 1, 1] = c; C[1, 0, 1] = c
    C[1, 1, 2] = 2.0 * d; C[0, 0, 2] = -d; C[2, 2, 2] = -d
    C[1, 2, 3] = c; C[2, 1, 3] = c
    C[2, 2, 4] = c; C[0, 0, 4] = -c
    return np.transpose(C, (2, 0, 1)).reshape(5, 9)


def _f32(x):
    return jnp.asarray(np.asarray(x, dtype=np.float32))


_EYE = np.eye
_ONE = np.ones
_RM24 = np.tile(_EYE(3), (1, 8))
_RB9 = np.tile(_EYE(9), (1, 8))
_RA64 = np.kron(_EYE(64), _ONE((1, 3)))
_RB24 = np.kron(_EYE(8), np.tile(_EYE(3), (1, 8)))
_RF192 = np.tile(_EYE(24), (8, 1))
_RA24 = np.kron(_EYE(24), _ONE((1, 3)))

_RSH2_NP = np.zeros((9, 96), dtype=np.float32)
_RSH2_NP[1:4, 0:24] = _RM24
_RSH2_NP[4:9, 24:96] = _np_c121() @ _RB9
_RSH2 = _f32(_RSH2_NP)

_REPBD_NP = np.zeros((72, 1536), dtype=np.float32)
_REPBD_NP[0:32, 0:1024] = np.kron(_EYE(32), _ONE((1, 32)))
_REPBD_NP[32:64, 1024:1280] = np.kron(_EYE(32), _ONE((1, 8)))
_REPBD_NP[64:72, 1280:1536] = np.kron(_EYE(8), _ONE((1, 32)))
_REPBD = _f32(_REPBD_NP)

_FOLDA_NP = np.zeros((1536, 40), dtype=np.float32)
_FOLDA_NP[0:1024, 0:32] = _N0 * np.tile(_EYE(32), (32, 1))
_FOLDA_NP[1024:1280, 32:40] = np.tile(_EYE(8), (32, 1))
_FOLDA_NP[1280:1536, 0:32] = _N0 * _I3 * np.tile(_EYE(32), (8, 1))
_FOLDA = _f32(_FOLDA_NP)

_RA2 = _f32(np.block([
    [_RA64, np.zeros((64, 192))], [np.zeros((64, 192)), _RA64]]))
_RB2 = _f32(np.block([
    [_RB24, np.zeros((24, 192))], [np.zeros((24, 192)), _RB24]]))
_FOLD35 = _f32(np.concatenate([_N1 * _I3 * _RF192, _N1 * _RF192], axis=0))

_RF248 = _f32(np.kron(_EYE(8), _ONE((3, 1))))
_RF72 = _f32(np.kron(_EYE(8), np.tile(_EYE(3), (3, 1))))
_RV24S = _f32(_N1 * _I3 * np.kron(_EYE(8), _ONE((1, 3))))


@functools.cache
def _build_sc_gather():
    @functools.partial(
        pl.kernel,
        mesh=plsc.VectorSubcoreMesh(core_axis_name="c", subcore_axis_name="s"),
        out_type=jax.ShapeDtypeStruct((E, D_PAIR), jnp.float32),
        scratch_types=[
            pltpu.VMEM((EPW,), jnp.int32),
            pltpu.VMEM((EPW,), jnp.int32),
            pltpu.VMEM((GC, CW), jnp.int32),
            pltpu.VMEM((GC, CW, D_PAIR), jnp.float32),
            pltpu.SemaphoreType.DMA,
        ],
    )
    def _sc_gather(ii_hbm, jj_hbm, table_hbm, out_hbm, ii_v, jj_v, idx_v,
                   rows_v, sem):
        wid = lax.axis_index("s") * 2 + lax.axis_index("c")
        base = wid * EPW
        pltpu.sync_copy(ii_hbm.at[pl.ds(base, EPW)], ii_v)
        pltpu.sync_copy(jj_hbm.at[pl.ds(base, EPW)], jj_v)
        for c in range(GC):
            for k in range(CW // 16):
                s = pl.ds(c * CW + k * 16, 16)
                idx_v[c, pl.ds(k * 16, 16)] = ii_v[s] * L + jj_v[s]
        copies = [
            pltpu.async_copy(table_hbm.at[idx_v.at[c]], rows_v.at[c], sem)
            for c in range(GC)
        ]
        for cp in copies:
            cp.wait()
        for c in range(GC):
            pltpu.sync_copy(rows_v.at[c],
                            out_hbm.at[pl.ds(base + c * CW, CW)])

    return _sc_gather


def _tc_body(ef_ref, src_ref, dst_ref, sh_ref, node_ref, l1f_ref, l1rep_ref,
             plw_ref, plb_ref, lng_ref, lnb_ref, f1w_ref, f1b_ref,
             wmw_ref, wmb_ref, pnw_ref, pnb_ref,
             rsh2_ref, rf248_ref, repbd_ref, folda_ref, rf72_ref,
             ra2_ref, rb2_ref, fold35_ref, rv24s_ref,
             nodeout_ref, l1o_ref,
             ng_s, acc_s, cnt_s):
    g = pl.program_id(0)

    @pl.when(g == 0)
    def _init():
        ng_s[:, 0:L0] = jnp.dot(node_ref[:], plw_ref[:],
                                preferred_element_type=jnp.float32) + plb_ref[:]
        ng_s[:, L0:56] = l1f_ref[:]
        ng_s[:, 56:128] = l1rep_ref[:]
        acc_s[:] = jnp.zeros_like(acc_s)
        cnt_s[:] = jnp.zeros_like(cnt_s)

    def md(a, b):
        return jnp.dot(a, b, preferred_element_type=jnp.float32)

    ef = ef_ref[:]
    mu = jnp.mean(ef, axis=1, keepdims=True)
    xc = ef - mu
    var = jnp.mean(xc * xc, axis=1, keepdims=True)
    h = xc * lax.rsqrt(var + 1e-5) * lng_ref[:] + lnb_ref[:]
    h = jnp.maximum(md(h, f1w_ref[:]) + f1b_ref[:], 0.0)

    wm = md(h, wmw_ref[:]) + wmb_ref[:]

    dstv = dst_ref[0]
    iota_n = lax.broadcasted_iota(jnp.int32, (BE, L), 1)
    ohd = (dstv == iota_n).astype(jnp.float32)
    xg = md(ohd, ng_s[:])
    xs = xg[:, 0:L0]
    xv = xg[:, L0:56]
    xva = xg[:, 56:128]

    sh = sh_ref[:]
    x2s = sh[:, 0:1]
    shx = md(sh, rsh2_ref[:])
    x2v24 = shx[:, 0:24]
    db = shx[:, 24:96]

    dotp = md(xv * x2v24, rf248_ref[:])
    lhs = jnp.concatenate([xs * x2s, xs, dotp], axis=1)
    rep = md(lhs, repbd_ref[:])
    oa = md(wm[:, 0:1536] * rep, folda_ref[:])
    o0 = oa[:, 0:32]
    t2 = oa[:, 32:40]

    tmp = md(xva * db, rf72_ref[:])
    a35 = md(wm[:, 1536:1664], ra2_ref[:])
    b35 = md(jnp.concatenate([xv * x2s, tmp], axis=1), rb2_ref[:])
    out1 = md(a35 * b35, fold35_ref[:])
    out1 = out1 + md(t2, rv24s_ref[:]) * x2v24

    conv = jnp.concatenate([o0, out1], axis=1)
    srow = src_ref[0]
    iota_t = lax.broadcasted_iota(jnp.int32, (L, BE), 0)
    ohsT = (srow == iota_t).astype(jnp.float32)
    acc_s[:] = acc_s[:] + md(ohsT, conv)
    cnt_s[:] = cnt_s[:] + jnp.sum(ohsT, axis=1, keepdims=True)

    @pl.when(g == G - 1)
    def _finish():
        cnt = jnp.maximum(cnt_s[:], 1.0)
        m = acc_s[:] / cnt
        nodeout_ref[:] = (jnp.dot(m[:, 0:L0], pnw_ref[:],
                                  preferred_element_type=jnp.float32)
                          + pnb_ref[:] + node_ref[:])
        l1o_ref[:] = m[:, L0:56] + l1f_ref[:]


def _rep(shape):
    nd = len(shape)
    return pl.BlockSpec(shape, lambda g, _n=nd: (0,) * _n)


def _build_tc(interpret: bool = False):
    in_specs = [
        pl.BlockSpec((BE, D_PAIR), lambda g: (g, 0)),
        pl.BlockSpec((1, 1, BE), lambda g: (g, 0, 0)),
        pl.BlockSpec((1, BE, 1), lambda g: (g, 0, 0)),
        pl.BlockSpec((BE, 9), lambda g: (g, 0)),
        _rep((L, D_NODE)),
        _rep((L, 3 * L1)),
        _rep((L, 72)),
        _rep((D_NODE, L0)), _rep((1, L0)),
        _rep((1, D_PAIR)), _rep((1, D_PAIR)),
        _rep((D_PAIR, D_PAIR)), _rep((1, D_PAIR)),
        _rep((D_PAIR, 1664)), _rep((1, 1664)),
        _rep((L0, D_NODE)), _rep((1, D_NODE)),
        _rep((9, 96)), _rep((24, 8)), _rep((72, 1536)),
        _rep((1536, 40)), _rep((72, 24)),
        _rep((128, 384)), _rep((48, 384)), _rep((384, 24)),
        _rep((8, 24)),
    ]
    out_specs = [
        pl.BlockSpec((L, D_NODE), lambda g: (0, 0)),
        pl.BlockSpec((L, 3 * L1), lambda g: (0, 0)),
    ]
    return pl.pallas_call(
        _tc_body,
        grid=(G,),
        in_specs=in_specs,
        out_specs=out_specs,
        out_shape=[
            jax.ShapeDtypeStruct((L, D_NODE), jnp.float32),
            jax.ShapeDtypeStruct((L, 3 * L1), jnp.float32),
        ],
        scratch_shapes=[
            pltpu.VMEM((L, 128), jnp.float32),
            pltpu.VMEM((L, 56), jnp.float32),
            pltpu.VMEM((L, 1), jnp.float32),
        ],
        compiler_params=pltpu.CompilerParams(
            dimension_semantics=("arbitrary",)),
        interpret=interpret,
    )


def _reorder_fc2(w):
    return jnp.concatenate(
        [w[:, 0:1280], w[:, 1344:1600], w[:, 1280:1344], w[:, 1600:1664]],
        axis=1)


def _prep_args(ef, node, l1_feats, edge_src, edge_dst, edge_sh,
               proj_l0_w, proj_l0_b, ln_g, ln_b, fc1_w, fc1_b, fc2_w, fc2_b,
               proj_node_w, proj_node_b):
    r1 = lambda a: a.reshape(1, -1)
    l1f2 = l1_feats.reshape(L, 3 * L1)
    return (
        ef,
        edge_src.reshape(G, 1, BE),
        edge_dst.reshape(G, BE, 1),
        edge_sh,
        node.reshape(L, D_NODE),
        l1f2,
        jnp.repeat(l1f2, 3, axis=1),
        proj_l0_w, r1(proj_l0_b), r1(ln_g), r1(ln_b),
        fc1_w, r1(fc1_b),
        _reorder_fc2(fc2_w), _reorder_fc2(fc2_b.reshape(1, -1)),
        proj_node_w, r1(proj_node_b),
        _RSH2, _RF248, _REPBD, _FOLDA, _RF72,
        _RA2, _RB2, _FOLD35, _RV24S,
    )


def kernel(node, pair, l1_feats, pair_index, edge_src, edge_dst, edge_sh,
           proj_l0_w, proj_l0_b, ln_g, ln_b, fc1_w, fc1_b, fc2_w, fc2_b,
           proj_node_w, proj_node_b):
    table = pair.reshape(L * L, D_PAIR)
    ef = _build_sc_gather()(pair_index[1], pair_index[2], table)
    args = _prep_args(ef, node, l1_feats, edge_src, edge_dst, edge_sh,
                      proj_l0_w, proj_l0_b, ln_g, ln_b, fc1_w, fc1_b,
                      fc2_w, fc2_b, proj_node_w, proj_node_b)
    node_out, l1o = _build_tc()(*args)
    return (node_out.reshape(1, L, D_NODE), l1o.reshape(1, L, 3 * L1))


# speedup vs baseline: 1.1082x; 1.0299x over previous
import functools

import jax
import jax.numpy as jnp
import numpy as np
from jax import lax
from jax.experimental import pallas as pl
from jax.experimental.pallas import tpu as pltpu
from jax.experimental.pallas import tpu_sc as plsc

L = 512
E = 16384
D_NODE = 256
D_PAIR = 128
L0 = 32
L1 = 8
BE = 2048
G = E // BE
NW = 32
EPW = E // NW
GC = 4
CW = EPW // GC

_N0 = 1.0 / np.sqrt(40.0)
_N1 = np.sqrt(3.0 / 48.0)
_I3 = 1.0 / np.sqrt(3.0)


def _np_c121():
    C = np.zeros((3, 3, 5), dtype=np.float32)
    c = 1.0 / np.sqrt(10.0)
    d = 1.0 / np.sqrt(30.0)
    C[0, 2, 0] = c; C[2, 0, 0] = c
    C[0, 1, 1] = c; C[1, 0, 1] = c
    C[1, 1, 2] = 2.0 * d; C[0, 0, 2] = -d; C[2, 2, 2] = -d
    C[1, 2, 3] = c; C[2, 1, 3] = c
    C[2, 2, 4] = c; C[0, 0, 4] = -c
    return np.transpose(C, (2, 0, 1)).reshape(5, 9)


def _f32(x):
    return jnp.asarray(np.asarray(x, dtype=np.float32))


_EYE = np.eye
_ONE = np.ones
_RM24 = np.tile(_EYE(3), (1, 8))
_RB9 = np.tile(_EYE(9), (1, 8))
_RA64 = np.kron(_EYE(64), _ONE((1, 3)))
_RB24 = np.kron(_EYE(8), np.tile(_EYE(3), (1, 8)))
_RF192 = np.tile(_EYE(24), (8, 1))
_RA24 = np.kron(_EYE(24), _ONE((1, 3)))

_RSH2_NP = np.zeros((9, 96), dtype=np.float32)
_RSH2_NP[1:4, 0:24] = _RM24
_RSH2_NP[4:9, 24:96] = _np_c121() @ _RB9
_RSH2 = _f32(_RSH2_NP)

def _bf16(x):
    return jnp.asarray(np.asarray(x, dtype=np.float32)).astype(jnp.bfloat16)


_REPBD_NP = np.zeros((72, 1536), dtype=np.float32)
_REPBD_NP[0:32, 0:1024] = np.kron(_EYE(32), _ONE((1, 32)))
_REPBD_NP[32:64, 1024:1280] = np.kron(_EYE(32), _ONE((1, 8)))
_REPBD_NP[64:72, 1280:1536] = np.kron(_EYE(8), _ONE((1, 32)))
_REPBD = _bf16(_REPBD_NP)

_FOLDA_NP = np.zeros((1536, 72), dtype=np.float32)
_FOLDA_NP[0:1024, 0:32] = np.tile(_EYE(32), (32, 1))
_FOLDA_NP[1024:1280, 32:40] = np.tile(_EYE(8), (32, 1))
_FOLDA_NP[1280:1536, 40:72] = np.tile(_EYE(32), (8, 1))
_FOLDA01 = _bf16(_FOLDA_NP)

_RA2 = _bf16(np.block([
    [_RA64, np.zeros((64, 192))], [np.zeros((64, 192)), _RA64]]))
_RB2 = _bf16(np.block([
    [_RB24, np.zeros((24, 192))], [np.zeros((24, 192)), _RB24]]))
_FOLD3501 = _bf16(np.block([
    [_RF192, np.zeros((192, 24))], [np.zeros((192, 24)), _RF192]]))

_RF248 = _f32(np.kron(_EYE(8), _ONE((3, 1))))
_RF72 = _f32(np.kron(_EYE(8), np.tile(_EYE(3), (3, 1))))
_RV24S = _f32(_N1 * _I3 * np.kron(_EYE(8), _ONE((1, 3))))


@functools.cache
def _build_sc_gather():
    @functools.partial(
        pl.kernel,
        mesh=plsc.VectorSubcoreMesh(core_axis_name="c", subcore_axis_name="s"),
        out_type=jax.ShapeDtypeStruct((E, D_PAIR), jnp.float32),
        scratch_types=[
            pltpu.VMEM((EPW,), jnp.int32),
            pltpu.VMEM((EPW,), jnp.int32),
            pltpu.VMEM((GC, CW), jnp.int32),
            pltpu.VMEM((GC, CW, D_PAIR), jnp.float32),
            pltpu.SemaphoreType.DMA,
        ],
    )
    def _sc_gather(ii_hbm, jj_hbm, table_hbm, out_hbm, ii_v, jj_v, idx_v,
                   rows_v, sem):
        wid = lax.axis_index("s") * 2 + lax.axis_index("c")
        base = wid * EPW
        pltpu.sync_copy(ii_hbm.at[pl.ds(base, EPW)], ii_v)
        pltpu.sync_copy(jj_hbm.at[pl.ds(base, EPW)], jj_v)
        for c in range(GC):
            for k in range(CW // 16):
                s = pl.ds(c * CW + k * 16, 16)
                idx_v[c, pl.ds(k * 16, 16)] = ii_v[s] * L + jj_v[s]
        copies = [
            pltpu.async_copy(table_hbm.at[idx_v.at[c]], rows_v.at[c], sem)
            for c in range(GC)
        ]
        for cp in copies:
            cp.wait()
        for c in range(GC):
            pltpu.sync_copy(rows_v.at[c],
                            out_hbm.at[pl.ds(base + c * CW, CW)])

    return _sc_gather


def _tc_body(ef_ref, src_ref, dst_ref, sh_ref, node_ref, l1f_ref, l1rep_ref,
             plw_ref, plb_ref, lng_ref, lnb_ref, f1w_ref, f1b_ref,
             wmw_ref, pnw_ref, pnb_ref,
             rsh2_ref, rf248_ref, repbd_ref, folda_ref, rf72_ref,
             ra2_ref, rb2_ref, fold35_ref, rv24s_ref, bbda_ref, bk35_ref,
             nodeout_ref, l1o_ref,
             ng_s, acc_s):
    g = pl.program_id(0)
    bf = jnp.bfloat16

    @pl.when(g == 0)
    def _init():
        l0 = jnp.dot(node_ref[:], plw_ref[:],
                     preferred_element_type=jnp.float32) + plb_ref[:]
        ng_s[:, 0:L0] = l0.astype(bf)
        ng_s[:, L0:56] = l1f_ref[:].astype(bf)
        ng_s[:, 56:128] = l1rep_ref[:].astype(bf)
        acc_s[:] = jnp.zeros_like(acc_s)

    def md(a, b):
        return jnp.dot(a, b, preferred_element_type=jnp.float32)

    def mdb(a, b):
        return jnp.dot(a, b, preferred_element_type=jnp.float32)

    ef = ef_ref[:]
    mu = jnp.mean(ef, axis=1, keepdims=True)
    xc = ef - mu
    var = jnp.mean(xc * xc, axis=1, keepdims=True)
    h = xc * lax.rsqrt(var + 1e-5) * lng_ref[:] + lnb_ref[:]
    h = jnp.maximum(md(h, f1w_ref[:]) + f1b_ref[:], 0.0)

    wm = mdb(h.astype(bf), wmw_ref[:])

    dstv = dst_ref[0]
    iota_n = lax.broadcasted_iota(jnp.int32, (BE, L), 1)
    ohd = (dstv == iota_n).astype(bf)
    xg = md(ohd, ng_s[:])
    xs = xg[:, 0:L0]
    xv = xg[:, L0:56]
    xva = xg[:, 56:128]

    sh = sh_ref[:]
    x2s = sh[:, 0:1]
    shx = md(sh, rsh2_ref[:])
    x2v24 = shx[:, 0:24]
    db = shx[:, 24:96]

    dotp = md(xv * x2v24, rf248_ref[:])
    lhs = jnp.concatenate([xs * x2s, xs, dotp], axis=1)
    rep = mdb(lhs.astype(bf), repbd_ref[:])
    fa = md((wm[:, 0:1536] * rep).astype(bf), folda_ref[:])
    obias = md(lhs, bbda_ref[:])
    o0 = _N0 * fa[:, 0:32] + (_N0 * _I3) * fa[:, 40:72] + obias[:, 0:32]
    t2 = fa[:, 32:40] + obias[:, 32:40]

    tmp = md(xva * db, rf72_ref[:])
    a35 = mdb(wm[:, 1536:1664].astype(bf), ra2_ref[:])
    blhs = jnp.concatenate([xv * x2s, tmp], axis=1)
    b35 = mdb(blhs.astype(bf), rb2_ref[:])
    f35 = md((a35 * b35).astype(bf), fold35_ref[:])
    out1 = (_N1 * _I3) * f35[:, 0:24] + _N1 * f35[:, 24:48] \
        + md(blhs, bk35_ref[:]) + md(t2, rv24s_ref[:]) * x2v24

    conv = jnp.concatenate(
        [o0, out1, jnp.ones((BE, 1), jnp.float32)], axis=1)
    srow = src_ref[0]
    iota_t = lax.broadcasted_iota(jnp.int32, (L, BE), 0)
    ohsT = (srow == iota_t).astype(bf)
    acc_s[:] = acc_s[:] + md(ohsT, conv.astype(bf))

    @pl.when(g == G - 1)
    def _finish():
        cnt = jnp.maximum(acc_s[:, 56:57], 1.0)
        m = acc_s[:, 0:56] / cnt
        nodeout_ref[:] = (jnp.dot(m[:, 0:L0], pnw_ref[:],
                                  preferred_element_type=jnp.float32)
                          + pnb_ref[:] + node_ref[:])
        l1o_ref[:] = m[:, L0:56] + l1f_ref[:]


def _rep(shape):
    nd = len(shape)
    return pl.BlockSpec(shape, lambda g, _n=nd: (0,) * _n)


def _build_tc(interpret: bool = False):
    in_specs = [
        pl.BlockSpec((BE, D_PAIR), lambda g: (g, 0)),
        pl.BlockSpec((1, 1, BE), lambda g: (g, 0, 0)),
        pl.BlockSpec((1, BE, 1), lambda g: (g, 0, 0)),
        pl.BlockSpec((BE, 9), lambda g: (g, 0)),
        _rep((L, D_NODE)),
        _rep((L, 3 * L1)),
        _rep((L, 72)),
        _rep((D_NODE, L0)), _rep((1, L0)),
        _rep((1, D_PAIR)), _rep((1, D_PAIR)),
        _rep((D_PAIR, D_PAIR)), _rep((1, D_PAIR)),
        _rep((D_PAIR, 1664)),
        _rep((L0, D_NODE)), _rep((1, D_NODE)),
        _rep((9, 96)), _rep((24, 8)), _rep((72, 1536)),
        _rep((1536, 72)), _rep((72, 24)),
        _rep((128, 384)), _rep((48, 384)), _rep((384, 48)),
        _rep((8, 24)), _rep((72, 40)), _rep((48, 24)),
    ]
    out_specs = [
        pl.BlockSpec((L, D_NODE), lambda g: (0, 0)),
        pl.BlockSpec((L, 3 * L1), lambda g: (0, 0)),
    ]
    return pl.pallas_call(
        _tc_body,
        grid=(G,),
        in_specs=in_specs,
        out_specs=out_specs,
        out_shape=[
            jax.ShapeDtypeStruct((L, D_NODE), jnp.float32),
            jax.ShapeDtypeStruct((L, 3 * L1), jnp.float32),
        ],
        scratch_shapes=[
            pltpu.VMEM((L, 128), jnp.bfloat16),
            pltpu.VMEM((L, 57), jnp.float32),
        ],
        compiler_params=pltpu.CompilerParams(
            dimension_semantics=("arbitrary",)),
        interpret=interpret,
    )


def _reorder_fc2(w):
    return jnp.concatenate(
        [w[:, 0:1280], w[:, 1344:1600], w[:, 1280:1344], w[:, 1600:1664]],
        axis=1)


def _prep_args(ef, node, l1_feats, edge_src, edge_dst, edge_sh,
               proj_l0_w, proj_l0_b, ln_g, ln_b, fc1_w, fc1_b, fc2_w, fc2_b,
               proj_node_w, proj_node_b):
    r1 = lambda a: a.reshape(1, -1)
    l1f2 = l1_feats.reshape(L, 3 * L1)
    eye3 = jnp.asarray(np.eye(3, dtype=np.float32))
    b1 = fc2_b[0:1024].reshape(L0, L0)
    b2 = fc2_b[1024:1280].reshape(L0, L1)
    b3 = fc2_b[1280:1344].reshape(L1, L1)
    b4 = fc2_b[1344:1600].reshape(L1, L0)
    b5 = fc2_b[1600:1664].reshape(L1, L1)
    z328 = jnp.zeros((L0, L1), jnp.float32)
    bbda = jnp.concatenate([
        jnp.concatenate([_N0 * b1, z328], axis=1),
        jnp.concatenate([jnp.zeros((L0, L0), jnp.float32), b2], axis=1),
        jnp.concatenate([_N0 * _I3 * b4, jnp.zeros((L1, L1), jnp.float32)],
                        axis=1),
    ], axis=0)
    bk35 = jnp.concatenate([_N1 * _I3 * jnp.kron(b3, eye3),
                            _N1 * jnp.kron(b5, eye3)], axis=0)
    return (
        ef,
        edge_src.reshape(G, 1, BE),
        edge_dst.reshape(G, BE, 1),
        edge_sh,
        node.reshape(L, D_NODE),
        l1f2,
        jnp.repeat(l1f2, 3, axis=1),
        proj_l0_w, r1(proj_l0_b), r1(ln_g), r1(ln_b),
        fc1_w, r1(fc1_b),
        _reorder_fc2(fc2_w).astype(jnp.bfloat16),
        proj_node_w, r1(proj_node_b),
        _RSH2, _RF248, _REPBD, _FOLDA01, _RF72,
        _RA2, _RB2, _FOLD3501, _RV24S, bbda, bk35,
    )


def kernel(node, pair, l1_feats, pair_index, edge_src, edge_dst, edge_sh,
           proj_l0_w, proj_l0_b, ln_g, ln_b, fc1_w, fc1_b, fc2_w, fc2_b,
           proj_node_w, proj_node_b):
    table = pair.reshape(L * L, D_PAIR)
    ef = _build_sc_gather()(pair_index[1], pair_index[2], table)
    args = _prep_args(ef, node, l1_feats, edge_src, edge_dst, edge_sh,
                      proj_l0_w, proj_l0_b, ln_g, ln_b, fc1_w, fc1_b,
                      fc2_w, fc2_b, proj_node_w, proj_node_b)
    node_out, l1o = _build_tc()(*args)
    return (node_out.reshape(1, L, D_NODE), l1o.reshape(1, L, 3 * L1))


# speedup vs baseline: 1.1092x; 1.0009x over previous
import functools

import jax
import jax.numpy as jnp
import numpy as np
from jax import lax
from jax.experimental import pallas as pl
from jax.experimental.pallas import tpu as pltpu
from jax.experimental.pallas import tpu_sc as plsc

L = 512
E = 16384
D_NODE = 256
D_PAIR = 128
L0 = 32
L1 = 8
BE = 2048
G = E // BE
NW = 32
EPW = E // NW
GC = 4
CW = EPW // GC

_N0 = 1.0 / np.sqrt(40.0)
_N1 = np.sqrt(3.0 / 48.0)
_I3 = 1.0 / np.sqrt(3.0)


def _np_c121():
    C = np.zeros((3, 3, 5), dtype=np.float32)
    c = 1.0 / np.sqrt(10.0)
    d = 1.0 / np.sqrt(30.0)
    C[0, 2, 0] = c; C[2, 0, 0] = c
    C[0, 1, 1] = c; C[1, 0, 1] = c
    C[1, 1, 2] = 2.0 * d; C[0, 0, 2] = -d; C[2, 2, 2] = -d
    C[1, 2, 3] = c; C[2, 1, 3] = c
    C[2, 2, 4] = c; C[0, 0, 4] = -c
    return np.transpose(C, (2, 0, 1)).reshape(5, 9)


def _f32(x):
    return jnp.asarray(np.asarray(x, dtype=np.float32))


_EYE = np.eye
_ONE = np.ones
_RM24 = np.tile(_EYE(3), (1, 8))
_RB9 = np.tile(_EYE(9), (1, 8))
_RA64 = np.kron(_EYE(64), _ONE((1, 3)))
_RB24 = np.kron(_EYE(8), np.tile(_EYE(3), (1, 8)))
_RF192 = np.tile(_EYE(24), (8, 1))
_RA24 = np.kron(_EYE(24), _ONE((1, 3)))

_RSH2_NP = np.zeros((9, 96), dtype=np.float32)
_RSH2_NP[1:4, 0:24] = _RM24
_RSH2_NP[4:9, 24:96] = _np_c121() @ _RB9
_RSH2 = _f32(_RSH2_NP)

def _bf16(x):
    return jnp.asarray(np.asarray(x, dtype=np.float32)).astype(jnp.bfloat16)


_REPBD_NP = np.zeros((72, 1536), dtype=np.float32)
_REPBD_NP[0:32, 0:1024] = np.kron(_EYE(32), _ONE((1, 32)))
_REPBD_NP[32:64, 1024:1280] = np.kron(_EYE(32), _ONE((1, 8)))
_REPBD_NP[64:72, 1280:1536] = np.kron(_EYE(8), _ONE((1, 32)))
_REPBD = _bf16(_REPBD_NP)

_FOLDA_NP = np.zeros((1536, 72), dtype=np.float32)
_FOLDA_NP[0:1024, 0:32] = np.tile(_EYE(32), (32, 1))
_FOLDA_NP[1024:1280, 32:40] = np.tile(_EYE(8), (32, 1))
_FOLDA_NP[1280:1536, 40:72] = np.tile(_EYE(32), (8, 1))
_FOLDA01 = _bf16(_FOLDA_NP)

_RA2 = _bf16(np.block([
    [_RA64, np.zeros((64, 192))], [np.zeros((64, 192)), _RA64]]))
_RB2 = _bf16(np.block([
    [_RB24, np.zeros((24, 192))], [np.zeros((24, 192)), _RB24]]))
_FOLD3501 = _bf16(np.block([
    [_RF192, np.zeros((192, 24))], [np.zeros((192, 24)), _RF192]]))

_RF248 = _f32(np.kron(_EYE(8), _ONE((3, 1))))
_RF72 = _f32(np.kron(_EYE(8), np.tile(_EYE(3), (3, 1))))
_RV24S = _f32(_N1 * _I3 * np.kron(_EYE(8), _ONE((1, 3))))


@functools.cache
def _build_sc_gather():
    @functools.partial(
        pl.kernel,
        mesh=plsc.VectorSubcoreMesh(core_axis_name="c", subcore_axis_name="s"),
        out_type=jax.ShapeDtypeStruct((E, D_PAIR), jnp.float32),
        scratch_types=[
            pltpu.VMEM((EPW,), jnp.int32),
            pltpu.VMEM((EPW,), jnp.int32),
            pltpu.VMEM((GC, CW), jnp.int32),
            pltpu.VMEM((GC, CW, D_PAIR), jnp.float32),
            pltpu.SemaphoreType.DMA,
        ],
    )
    def _sc_gather(ii_hbm, jj_hbm, table_hbm, out_hbm, ii_v, jj_v, idx_v,
                   rows_v, sem):
        wid = lax.axis_index("s") * 2 + lax.axis_index("c")
        base = wid * EPW
        pltpu.sync_copy(ii_hbm.at[pl.ds(base, EPW)], ii_v)
        pltpu.sync_copy(jj_hbm.at[pl.ds(base, EPW)], jj_v)
        for c in range(GC):
            for k in range(CW // 16):
                s = pl.ds(c * CW + k * 16, 16)
                idx_v[c, pl.ds(k * 16, 16)] = ii_v[s] * L + jj_v[s]
        copies = [
            pltpu.async_copy(table_hbm.at[idx_v.at[c]], rows_v.at[c], sem)
            for c in range(GC)
        ]
        for cp in copies:
            cp.wait()
        for c in range(GC):
            pltpu.sync_copy(rows_v.at[c],
                            out_hbm.at[pl.ds(base + c * CW, CW)])

    return _sc_gather


def _tc_body(ef_ref, src_ref, dst_ref, sh_ref, node_ref, l1f_ref, l1rep_ref,
             plw_ref, plb_ref, lng_ref, lnb_ref, f1w_ref, f1b_ref,
             wmw_ref, pnw_ref, pnb_ref,
             rsh2_ref, rf248_ref, repbd_ref, folda_ref, rf72_ref,
             ra2_ref, rb2_ref, fold35_ref, rv24s_ref, bbda_ref, bk35_ref,
             nodeout_ref, l1o_ref,
             ng_s, acc_s):
    g = pl.program_id(0)
    bf = jnp.bfloat16

    @pl.when(g == 0)
    def _init():
        l0 = jnp.dot(node_ref[:], plw_ref[:],
                     preferred_element_type=jnp.float32) + plb_ref[:]
        ng_s[:, 0:L0] = l0.astype(bf)
        ng_s[:, L0:56] = l1f_ref[:].astype(bf)
        ng_s[:, 56:128] = l1rep_ref[:].astype(bf)
        acc_s[:] = jnp.zeros_like(acc_s)

    def md(a, b):
        return jnp.dot(a, b, preferred_element_type=jnp.float32)

    def mdb(a, b):
        return jnp.dot(a, b, preferred_element_type=jnp.float32)

    ef = ef_ref[:]
    mu = jnp.mean(ef, axis=1, keepdims=True)
    xc = ef - mu
    var = jnp.mean(xc * xc, axis=1, keepdims=True)
    h = xc * lax.rsqrt(var + 1e-5) * lng_ref[:] + lnb_ref[:]
    h = jnp.maximum(mdb(h.astype(bf), f1w_ref[:]) + f1b_ref[:], 0.0)

    wm = mdb(h.astype(bf), wmw_ref[:])

    dstv = dst_ref[0]
    iota_n = lax.broadcasted_iota(jnp.int32, (BE, L), 1)
    ohd = (dstv == iota_n).astype(bf)
    xg = md(ohd, ng_s[:])
    xs = xg[:, 0:L0]
    xv = xg[:, L0:56]
    xva = xg[:, 56:128]

    sh = sh_ref[:]
    x2s = sh[:, 0:1]
    shx = mdb(sh.astype(bf), rsh2_ref[:])
    x2v24 = shx[:, 0:24]
    db = shx[:, 24:96]

    dotp = mdb((xv * x2v24).astype(bf), rf248_ref[:])
    lhs_bf = jnp.concatenate([xs * x2s, xs, dotp],
                             axis=1).astype(bf)
    rep = mdb(lhs_bf, repbd_ref[:])
    fa = md((wm[:, 0:1536] * rep).astype(bf), folda_ref[:])
    obias = mdb(lhs_bf, bbda_ref[:])
    o0 = _N0 * fa[:, 0:32] + (_N0 * _I3) * fa[:, 40:72] + obias[:, 0:32]
    t2 = fa[:, 32:40] + obias[:, 32:40]

    tmp = mdb((xva * db).astype(bf), rf72_ref[:])
    a35 = mdb(wm[:, 1536:1664].astype(bf), ra2_ref[:])
    blhs_bf = jnp.concatenate([xv * x2s, tmp], axis=1).astype(bf)
    b35 = mdb(blhs_bf, rb2_ref[:])
    f35 = md((a35 * b35).astype(bf), fold35_ref[:])
    out1 = (_N1 * _I3) * f35[:, 0:24] + _N1 * f35[:, 24:48] \
        + mdb(blhs_bf, bk35_ref[:]) \
        + mdb(t2.astype(bf), rv24s_ref[:]) * x2v24

    conv = jnp.concatenate(
        [o0, out1, jnp.ones((BE, 1), jnp.float32)], axis=1)
    srow = src_ref[0]
    iota_t = lax.broadcasted_iota(jnp.int32, (L, BE), 0)
    ohsT = (srow == iota_t).astype(bf)
    acc_s[:] = acc_s[:] + md(ohsT, conv.astype(bf))

    @pl.when(g == G - 1)
    def _finish():
        cnt = jnp.maximum(acc_s[:, 56:57], 1.0)
        m = acc_s[:, 0:56] / cnt
        nodeout_ref[:] = (jnp.dot(m[:, 0:L0], pnw_ref[:],
                                  preferred_element_type=jnp.float32)
                          + pnb_ref[:] + node_ref[:])
        l1o_ref[:] = m[:, L0:56] + l1f_ref[:]


def _rep(shape):
    nd = len(shape)
    return pl.BlockSpec(shape, lambda g, _n=nd: (0,) * _n)


def _build_tc(interpret: bool = False):
    in_specs = [
        pl.BlockSpec((BE, D_PAIR), lambda g: (g, 0)),
        pl.BlockSpec((1, 1, BE), lambda g: (g, 0, 0)),
        pl.BlockSpec((1, BE, 1), lambda g: (g, 0, 0)),
        pl.BlockSpec((BE, 9), lambda g: (g, 0)),
        _rep((L, D_NODE)),
        _rep((L, 3 * L1)),
        _rep((L, 72)),
        _rep((D_NODE, L0)), _rep((1, L0)),
        _rep((1, D_PAIR)), _rep((1, D_PAIR)),
        _rep((D_PAIR, D_PAIR)), _rep((1, D_PAIR)),
        _rep((D_PAIR, 1664)),
        _rep((L0, D_NODE)), _rep((1, D_NODE)),
        _rep((9, 96)), _rep((24, 8)), _rep((72, 1536)),
        _rep((1536, 72)), _rep((72, 24)),
        _rep((128, 384)), _rep((48, 384)), _rep((384, 48)),
        _rep((8, 24)), _rep((72, 40)), _rep((48, 24)),
    ]
    out_specs = [
        pl.BlockSpec((L, D_NODE), lambda g: (0, 0)),
        pl.BlockSpec((L, 3 * L1), lambda g: (0, 0)),
    ]
    return pl.pallas_call(
        _tc_body,
        grid=(G,),
        in_specs=in_specs,
        out_specs=out_specs,
        out_shape=[
            jax.ShapeDtypeStruct((L, D_NODE), jnp.float32),
            jax.ShapeDtypeStruct((L, 3 * L1), jnp.float32),
        ],
        scratch_shapes=[
            pltpu.VMEM((L, 128), jnp.bfloat16),
            pltpu.VMEM((L, 57), jnp.float32),
        ],
        compiler_params=pltpu.CompilerParams(
            dimension_semantics=("arbitrary",)),
        interpret=interpret,
    )


def _reorder_fc2(w):
    return jnp.concatenate(
        [w[:, 0:1280], w[:, 1344:1600], w[:, 1280:1344], w[:, 1600:1664]],
        axis=1)


def _prep_args(ef, node, l1_feats, edge_src, edge_dst, edge_sh,
               proj_l0_w, proj_l0_b, ln_g, ln_b, fc1_w, fc1_b, fc2_w, fc2_b,
               proj_node_w, proj_node_b):
    r1 = lambda a: a.reshape(1, -1)
    l1f2 = l1_feats.reshape(L, 3 * L1)
    eye3 = jnp.asarray(np.eye(3, dtype=np.float32))
    b1 = fc2_b[0:1024].reshape(L0, L0)
    b2 = fc2_b[1024:1280].reshape(L0, L1)
    b3 = fc2_b[1280:1344].reshape(L1, L1)
    b4 = fc2_b[1344:1600].reshape(L1, L0)
    b5 = fc2_b[1600:1664].reshape(L1, L1)
    z328 = jnp.zeros((L0, L1), jnp.float32)
    bbda = jnp.concatenate([
        jnp.concatenate([_N0 * b1, z328], axis=1),
        jnp.concatenate([jnp.zeros((L0, L0), jnp.float32), b2], axis=1),
        jnp.concatenate([_N0 * _I3 * b4, jnp.zeros((L1, L1), jnp.float32)],
                        axis=1),
    ], axis=0)
    bk35 = jnp.concatenate([_N1 * _I3 * jnp.kron(b3, eye3),
                            _N1 * jnp.kron(b5, eye3)], axis=0)
    return (
        ef,
        edge_src.reshape(G, 1, BE),
        edge_dst.reshape(G, BE, 1),
        edge_sh,
        node.reshape(L, D_NODE),
        l1f2,
        jnp.repeat(l1f2, 3, axis=1),
        proj_l0_w, r1(proj_l0_b), r1(ln_g), r1(ln_b),
        fc1_w.astype(jnp.bfloat16), r1(fc1_b),
        _reorder_fc2(fc2_w).astype(jnp.bfloat16),
        proj_node_w, r1(proj_node_b),
        _RSH2.astype(jnp.bfloat16), _RF248.astype(jnp.bfloat16),
        _REPBD, _FOLDA01, _RF72.astype(jnp.bfloat16),
        _RA2, _RB2, _FOLD3501, _RV24S.astype(jnp.bfloat16),
        bbda.astype(jnp.bfloat16), bk35.astype(jnp.bfloat16),
    )


def kernel(node, pair, l1_feats, pair_index, edge_src, edge_dst, edge_sh,
           proj_l0_w, proj_l0_b, ln_g, ln_b, fc1_w, fc1_b, fc2_w, fc2_b,
           proj_node_w, proj_node_b):
    table = pair.reshape(L * L, D_PAIR)
    ef = _build_sc_gather()(pair_index[1], pair_index[2], table)
    args = _prep_args(ef, node, l1_feats, edge_src, edge_dst, edge_sh,
                      proj_l0_w, proj_l0_b, ln_g, ln_b, fc1_w, fc1_b,
                      fc2_w, fc2_b, proj_node_w, proj_node_b)
    node_out, l1o = _build_tc()(*args)
    return (node_out.reshape(1, L, D_NODE), l1o.reshape(1, L, 3 * L1))
